# baseline (speedup 1.0000x reference)
import functools

import jax
import jax.numpy as jnp
from jax import lax
from jax.experimental import pallas as pl
from jax.experimental.pallas import tpu as pltpu
from jax.experimental.pallas import tpu_sc as plsc

NC = 2
NS = 16
NW = NC * NS
L = 16

B = 16384
EMB = 64
HIST = 200
BPW = B // NW
NCHUNK = BPW // 128


def _body(gidx_hbm, aidx_hbm, sidx_hbm, song_hbm, genre_hbm, artist_hbm,
          wb_hbm, out_hbm,
          sidx_v, pidx_v, cidx_v, rows_v, grows_v, arows_v, wv, outv,
          sem_s, sem_c):
    c = lax.axis_index("c")
    s = lax.axis_index("s")
    wid = s * NC + c
    base = wid * BPW

    pltpu.sync_copy(sidx_hbm.at[pl.ds(base, BPW)], sidx_v)
    for i in range(BPW // L):
        v = sidx_v[pl.ds(i * L, L)]
        pidx_v[i // 8, pl.ds((i % 8) * L, L)] = lax.shift_right_logical(v, 1)
    song_cps = [
        pltpu.async_copy(song_hbm.at[pidx_v.at[j]],
                         rows_v.at[pl.ds(j * 128, 128)], sem_s)
        for j in range(NCHUNK)
    ]

    pltpu.sync_copy(gidx_hbm.at[pl.ds(0, 128)], cidx_v.at[0])
    pltpu.sync_copy(gidx_hbm.at[pl.ds(128, 72)], cidx_v.at[1, pl.ds(0, 72)])
    pltpu.sync_copy(aidx_hbm.at[pl.ds(0, 128)], cidx_v.at[2])
    pltpu.sync_copy(aidx_hbm.at[pl.ds(128, 72)], cidx_v.at[3, pl.ds(0, 72)])
    const_cps = [
        pltpu.async_copy(genre_hbm.at[cidx_v.at[0]],
                         grows_v.at[pl.ds(0, 128)], sem_c),
        pltpu.async_copy(genre_hbm.at[cidx_v.at[1, pl.ds(0, 72)]],
                         grows_v.at[pl.ds(128, 72)], sem_c),
        pltpu.async_copy(artist_hbm.at[cidx_v.at[2]],
                         arows_v.at[pl.ds(0, 128)], sem_c),
        pltpu.async_copy(artist_hbm.at[cidx_v.at[3, pl.ds(0, 72)]],
                         arows_v.at[pl.ds(128, 72)], sem_c),
    ]

    pltpu.sync_copy(wb_hbm, wv)

    for cp in const_cps:
        cp.wait()

    zeros = jnp.zeros((L,), jnp.float32)
    lane = lax.iota(jnp.int32, L)

    dnums = lax.GatherDimensionNumbers(
        offset_dims=(), collapsed_slice_dims=(0,), start_index_map=(0,))

    def lperm(v, idx):
        return lax.gather(v, idx[:, None], dnums, slice_sizes=(1,),
                          mode=lax.GatherScatterMode.PROMISE_IN_BOUNDS)

    def allsum(v):
        for step in (1, 2, 4, 8):
            v = v + lperm(v, lane ^ step)
        return v

    def accum(rows_ref):
        def it(r, accs):
            return tuple(accs[k] + rows_ref[r, pl.ds(16 * k, 16)]
                         for k in range(4))
        return lax.fori_loop(0, HIST, it, (zeros,) * 4)

    gsum = accum(grows_v)
    asum = accum(arows_v)

    wg = [wv[0, pl.ds(16 * k, 16)] for k in range(4)]
    wa = [wv[0, pl.ds(64 + 16 * k, 16)] for k in range(4)]
    ws = [wv[0, pl.ds(128 + 16 * k, 16)] for k in range(4)]
    tg = gsum[0] * wg[0] + gsum[1] * wg[1] + gsum[2] * wg[2] + gsum[3] * wg[3]
    ta = asum[0] * wa[0] + asum[1] * wa[1] + asum[2] * wa[2] + asum[3] * wa[3]
    bias = allsum(wv[0, pl.ds(192, 16)])
    cconst = (allsum(tg) + allsum(ta)) * (1.0 / HIST) + bias

    for cp in song_cps:
        cp.wait()

    def group(g, _):
        par16 = (sidx_v[pl.ds(g * L, L)] & 1) * EMB
        acc = zeros
        for r in range(L):
            row = jnp.full((L,), g * L + r, jnp.int32)
            cbase = lperm(par16, jnp.full((L,), r, jnp.int32)) + lane
            v = (plsc.load_gather(rows_v, [row, cbase]) * ws[0]
                 + plsc.load_gather(rows_v, [row, cbase + 16]) * ws[1]
                 + plsc.load_gather(rows_v, [row, cbase + 32]) * ws[2]
                 + plsc.load_gather(rows_v, [row, cbase + 48]) * ws[3])
            acc = jnp.where(lane == r, allsum(v), acc)
        outv[pl.ds(g * L, L)] = acc + cconst
        return 0

    lax.fori_loop(0, BPW // L, group, 0)

    pltpu.sync_copy(outv, out_hbm.at[pl.ds(base, BPW)])


@jax.jit
def _run(gidx, aidx, sidx, song_table, genre_table, artist_table, wb):
    mesh = plsc.VectorSubcoreMesh(core_axis_name="c", subcore_axis_name="s",
                                  num_cores=NC, num_subcores=NS)
    return pl.kernel(
        _body,
        out_type=jax.ShapeDtypeStruct((B,), jnp.float32),
        mesh=mesh,
        scratch_types=[
            pltpu.VMEM((BPW,), jnp.int32),
            pltpu.VMEM((NCHUNK, 128), jnp.int32),
            pltpu.VMEM((4, 128), jnp.int32),
            pltpu.VMEM((BPW, 2 * EMB), jnp.float32),
            pltpu.VMEM((HIST, EMB), jnp.float32),
            pltpu.VMEM((HIST, EMB), jnp.float32),
            pltpu.VMEM((1, 208), jnp.float32),
            pltpu.VMEM((BPW,), jnp.float32),
            pltpu.SemaphoreType.DMA,
            pltpu.SemaphoreType.DMA,
        ],
        compiler_params=pltpu.CompilerParams(use_tc_tiling_on_sc=False,
                                             needs_layout_passes=False),
    )(gidx, aidx, sidx, song_table, genre_table, artist_table, wb)


def kernel(genre_indices, artist_indices, song_indices, song_table,
           genre_table, artist_table, fc_w, fc_b):
    wb = jnp.pad(jnp.concatenate([fc_w.reshape(-1), fc_b.reshape(-1)]),
                 (0, 15)).reshape(1, 208)
    song2 = song_table.reshape(-1, 2 * EMB)
    return _run(genre_indices.astype(jnp.int32),
                artist_indices.astype(jnp.int32),
                song_indices.astype(jnp.int32),
                song2, genre_table, artist_table, wb)


# speedup vs baseline: 5.4778x; 5.4778x over previous
import functools

import jax
import jax.numpy as jnp
from jax import lax
from jax.experimental import pallas as pl
from jax.experimental.pallas import tpu as pltpu
from jax.experimental.pallas import tpu_sc as plsc

NC = 2
NS = 16
NW = NC * NS
L = 16

B = 16384
EMB = 64
HIST = 200
BPW = B // NW
NCHUNK = BPW // 128


def _mv_body(x_ref, w_ref, o_ref):
    o_ref[...] = jnp.sum(x_ref[...] * w_ref[...], axis=0)


def _matvec(xt, w, blk):
    n = xt.shape[1]
    grid = (n + blk - 1) // blk
    return pl.pallas_call(
        _mv_body,
        grid=(grid,),
        in_specs=[
            pl.BlockSpec((EMB, blk), lambda i: (0, i)),
            pl.BlockSpec((EMB, 1), lambda i: (0, 0)),
        ],
        out_specs=pl.BlockSpec((blk,), lambda i: (i,)),
        out_shape=jax.ShapeDtypeStruct((n,), jnp.float32),
    )(xt, w)


def _sc_body(gidx_hbm, aidx_hbm, sidx_hbm, ss_hbm, gs_hbm, as_hbm, b16_hbm,
             out_hbm, sidx_v, cidx_v, sval_v, gval_v, aval_v, bv, outv,
             sem_s, sem_c):
    c = lax.axis_index("c")
    s = lax.axis_index("s")
    wid = s * NC + c
    base = wid * BPW

    for j in range(NCHUNK):
        pltpu.sync_copy(sidx_hbm.at[pl.ds(base + j * 128, 128)], sidx_v.at[j])
    song_cps = [
        pltpu.async_copy(ss_hbm.at[sidx_v.at[j]],
                         sval_v.at[pl.ds(j * 128, 128)], sem_s)
        for j in range(NCHUNK)
    ]

    pltpu.sync_copy(gidx_hbm.at[pl.ds(0, 128)], cidx_v.at[0])
    pltpu.sync_copy(gidx_hbm.at[pl.ds(128, 72)], cidx_v.at[1, pl.ds(0, 72)])
    pltpu.sync_copy(aidx_hbm.at[pl.ds(0, 128)], cidx_v.at[2])
    pltpu.sync_copy(aidx_hbm.at[pl.ds(128, 72)], cidx_v.at[3, pl.ds(0, 72)])

    zeros = jnp.zeros((L,), jnp.float32)
    gval_v[pl.ds(192, L)] = zeros
    aval_v[pl.ds(192, L)] = zeros

    const_cps = [
        pltpu.async_copy(gs_hbm.at[cidx_v.at[0]],
                         gval_v.at[pl.ds(0, 128)], sem_c),
        pltpu.async_copy(gs_hbm.at[cidx_v.at[1, pl.ds(0, 72)]],
                         gval_v.at[pl.ds(128, 72)], sem_c),
        pltpu.async_copy(as_hbm.at[cidx_v.at[2]],
                         aval_v.at[pl.ds(0, 128)], sem_c),
        pltpu.async_copy(as_hbm.at[cidx_v.at[3, pl.ds(0, 72)]],
                         aval_v.at[pl.ds(128, 72)], sem_c),
    ]
    pltpu.sync_copy(b16_hbm, bv)

    lane = lax.iota(jnp.int32, L)
    dnums = lax.GatherDimensionNumbers(
        offset_dims=(), collapsed_slice_dims=(0,), start_index_map=(0,))

    def allsum(v):
        for step in (1, 2, 4, 8):
            p = lax.gather(v, (lane ^ step)[:, None], dnums, slice_sizes=(1,),
                           mode=lax.GatherScatterMode.PROMISE_IN_BOUNDS)
            v = v + p
        return v

    for cp in const_cps:
        cp.wait()

    gtot = zeros
    atot = zeros
    for t in range(13):
        gtot = gtot + gval_v[pl.ds(t * L, L)]
        atot = atot + aval_v[pl.ds(t * L, L)]
    cconst = (allsum(gtot) + allsum(atot)) * (1.0 / HIST) + allsum(bv[...])

    for cp in song_cps:
        cp.wait()

    def group(g, _):
        outv[pl.ds(g * L, L)] = sval_v[pl.ds(g * L, L)] + cconst
        return 0

    lax.fori_loop(0, BPW // L, group, 0)

    pltpu.sync_copy(outv, out_hbm.at[pl.ds(base, BPW)])


@jax.jit
def _run(gidx, aidx, sidx, song_scores, genre_scores, artist_scores, b16):
    mesh = plsc.VectorSubcoreMesh(core_axis_name="c", subcore_axis_name="s",
                                  num_cores=NC, num_subcores=NS)
    return pl.kernel(
        _sc_body,
        out_type=jax.ShapeDtypeStruct((B,), jnp.float32),
        mesh=mesh,
        scratch_types=[
            pltpu.VMEM((NCHUNK, 128), jnp.int32),
            pltpu.VMEM((4, 128), jnp.int32),
            pltpu.VMEM((BPW,), jnp.float32),
            pltpu.VMEM((208,), jnp.float32),
            pltpu.VMEM((208,), jnp.float32),
            pltpu.VMEM((L,), jnp.float32),
            pltpu.VMEM((BPW,), jnp.float32),
            pltpu.SemaphoreType.DMA,
            pltpu.SemaphoreType.DMA,
        ],
        compiler_params=pltpu.CompilerParams(needs_layout_passes=False),
    )(gidx, aidx, sidx, song_scores, genre_scores, artist_scores, b16)


def kernel(genre_indices, artist_indices, song_indices, song_table,
           genre_table, artist_table, fc_w, fc_b):
    wg = fc_w[0, :EMB].reshape(EMB, 1)
    wa = fc_w[0, EMB:2 * EMB].reshape(EMB, 1)
    ws = fc_w[0, 2 * EMB:].reshape(EMB, 1)
    song_scores = _matvec(song_table.T, ws, 32768)
    genre_scores = _matvec(genre_table.T, wg, 1024)
    artist_scores = _matvec(artist_table.T, wa, 16384)
    b16 = jnp.pad(fc_b.reshape(-1), (0, L - 1))
    return _run(genre_indices.astype(jnp.int32),
                artist_indices.astype(jnp.int32),
                song_indices.astype(jnp.int32),
                song_scores, genre_scores, artist_scores, b16)
